# baseline (speedup 1.0000x reference)
import functools

import jax
import jax.numpy as jnp
from jax import lax
from jax.experimental import pallas as pl
from jax.experimental.pallas import tpu as pltpu
from jax.experimental.pallas import tpu_sc as plsc

_WORKER = 1000
_TASK = 20000
_ET = 2
_AB = 128
_WPAD = 1024

_mesh = plsc.VectorSubcoreMesh(core_axis_name="c", subcore_axis_name="s")


def _tc_body(b_ref, wf_ref, w_ref, lp2_ref, d_ref):
    z = jnp.dot(wf_ref[...], w_ref[...],
                preferred_element_type=jnp.float32) + b_ref[0]
    s = jax.nn.sigmoid(z)
    lp1 = jnp.maximum(jnp.log(s), -1e10)
    lp2 = jnp.maximum(jnp.log(1.0 - s), -1e10)
    lp2_ref[...] = jnp.broadcast_to(lp2, (_WPAD, 16))
    d_ref[...] = jnp.broadcast_to(lp1 - lp2, (_WPAD, 16))


@functools.partial(
    pl.kernel,
    mesh=_mesh,
    out_type=jax.ShapeDtypeStruct((_WORKER, _ET, _TASK), jnp.float32),
    scratch_types=[
        pltpu.VMEM((_ET, _TASK), jnp.float32),
        pltpu.VMEM((32, 16), jnp.float32),
        pltpu.VMEM((32, 16), jnp.float32),
        pltpu.VMEM((_ET, _TASK), jnp.float32),
        pltpu.VMEM((_ET, _TASK), jnp.float32),
        pltpu.SemaphoreType.DMA,
        pltpu.SemaphoreType.DMA,
    ],
)
def _sc_body(tau_hbm, lp2_hbm, d_hbm, out_hbm, tau_v, lp2_v, d_v, bufa, bufb,
             sema, semb):
    wid = lax.axis_index("s") * 2 + lax.axis_index("c")
    start = 8 * (wid * 4 - jnp.maximum(wid - 29, 0))
    cnt = jnp.where(wid < 29, 32, 24)
    pltpu.sync_copy(tau_hbm, tau_v)
    pltpu.sync_copy(lp2_hbm.at[pl.ds(start, 32)], lp2_v)
    pltpu.sync_copy(d_hbm.at[pl.ds(start, 32)], d_v)

    def compute_row(w, buf):
        lp2v = lp2_v[w]
        dv = d_v[w]

        @plsc.parallel_loop(0, _TASK // 16, unroll=16)
        def chunk(c):
            o = c * 16
            buf[0, pl.ds(o, 16)] = jnp.exp(lp2v + tau_v[0, pl.ds(o, 16)] * dv)
            buf[1, pl.ds(o, 16)] = jnp.exp(lp2v + tau_v[1, pl.ds(o, 16)] * dv)

    def pair(p, carry):
        w0 = 2 * p
        compute_row(w0, bufa)
        ca = pltpu.make_async_copy(bufa, out_hbm.at[start + w0], sema)
        ca.start()
        compute_row(w0 + 1, bufb)
        cb = pltpu.make_async_copy(bufb, out_hbm.at[start + w0 + 1], semb)
        cb.start()
        ca.wait()
        cb.wait()
        return carry

    lax.fori_loop(0, cnt // 2, pair, 0)


def kernel(inputs, W, b, P0):
    wf = jnp.pad(inputs[:_WORKER], ((0, _WPAD - _WORKER), (0, 0)))
    tau = inputs[_WORKER:, :_ET].T
    lp2b, db = pl.pallas_call(
        _tc_body,
        in_specs=[
            pl.BlockSpec(memory_space=pltpu.SMEM),
            pl.BlockSpec((_WPAD, _AB), lambda: (0, 0)),
            pl.BlockSpec((_AB, 1), lambda: (0, 0)),
        ],
        out_specs=[
            pl.BlockSpec((_WPAD, 16), lambda: (0, 0)),
            pl.BlockSpec((_WPAD, 16), lambda: (0, 0)),
        ],
        out_shape=[
            jax.ShapeDtypeStruct((_WPAD, 16), jnp.float32),
            jax.ShapeDtypeStruct((_WPAD, 16), jnp.float32),
        ],
    )(b, wf, W)
    out = _sc_body(tau, lp2b, db)
    return out.transpose(0, 2, 1)


# speedup vs baseline: 1.2201x; 1.2201x over previous
import functools

import jax
import jax.numpy as jnp
from jax import lax
from jax.experimental import pallas as pl
from jax.experimental.pallas import tpu as pltpu
from jax.experimental.pallas import tpu_sc as plsc

_WORKER = 1000
_TASK = 20000
_ET = 2
_AB = 128
_WPAD = 1024

_mesh = plsc.VectorSubcoreMesh(core_axis_name="c", subcore_axis_name="s")


def _tc_body(b_ref, wf_ref, w_ref, lp2_ref, d_ref):
    z = jnp.dot(wf_ref[...], w_ref[...],
                preferred_element_type=jnp.float32) + b_ref[0]
    s = jax.nn.sigmoid(z)
    lp1 = jnp.maximum(jnp.log(s), -1e10)
    lp2 = jnp.maximum(jnp.log(1.0 - s), -1e10)
    lp2_ref[...] = jnp.broadcast_to(lp2, (_WPAD, 16))
    d_ref[...] = jnp.broadcast_to(lp1 - lp2, (_WPAD, 16))


@functools.partial(
    pl.kernel,
    mesh=_mesh,
    out_type=jax.ShapeDtypeStruct((_WORKER, _ET, _TASK), jnp.float32),
    scratch_types=[
        pltpu.VMEM((_ET, _TASK), jnp.float32),
        pltpu.VMEM((32, 16), jnp.float32),
        pltpu.VMEM((32, 16), jnp.float32),
        pltpu.VMEM((_ET, _TASK), jnp.float32),
        pltpu.VMEM((_ET, _TASK), jnp.float32),
        pltpu.SemaphoreType.DMA,
        pltpu.SemaphoreType.DMA,
    ],
)
def _sc_body(tau_hbm, lp2_hbm, d_hbm, out_hbm, tau_v, lp2_v, d_v, bufa, bufb,
             sema, semb):
    wid = lax.axis_index("s") * 2 + lax.axis_index("c")
    start = 8 * (wid * 4 - jnp.maximum(wid - 29, 0))
    cnt = jnp.where(wid < 29, 32, 24)
    pltpu.sync_copy(tau_hbm, tau_v)
    pltpu.sync_copy(lp2_hbm.at[pl.ds(start, 32)], lp2_v)
    pltpu.sync_copy(d_hbm.at[pl.ds(start, 32)], d_v)

    def compute_row(w, buf):
        lp2v = lp2_v[w]
        dv = d_v[w]

        @plsc.parallel_loop(0, _TASK // 16, unroll=8)
        def chunk(c):
            o = c * 16
            buf[0, pl.ds(o, 16)] = jnp.exp(lp2v + tau_v[0, pl.ds(o, 16)] * dv)
            buf[1, pl.ds(o, 16)] = jnp.exp(lp2v + tau_v[1, pl.ds(o, 16)] * dv)

    def pair(p, carry):
        w0 = 2 * p

        @pl.when(p > 0)
        def _():
            pltpu.make_async_copy(bufa, out_hbm.at[start + w0 - 2], sema).wait()

        compute_row(w0, bufa)
        pltpu.make_async_copy(bufa, out_hbm.at[start + w0], sema).start()

        @pl.when(p > 0)
        def _():
            pltpu.make_async_copy(bufb, out_hbm.at[start + w0 - 1], semb).wait()

        compute_row(w0 + 1, bufb)
        pltpu.make_async_copy(bufb, out_hbm.at[start + w0 + 1], semb).start()
        return carry

    lax.fori_loop(0, cnt // 2, pair, 0)
    pltpu.make_async_copy(bufa, out_hbm.at[start + cnt - 2], sema).wait()
    pltpu.make_async_copy(bufb, out_hbm.at[start + cnt - 1], semb).wait()


def kernel(inputs, W, b, P0):
    wf = jnp.pad(inputs[:_WORKER], ((0, _WPAD - _WORKER), (0, 0)))
    tau = inputs[_WORKER:, :_ET].T
    lp2b, db = pl.pallas_call(
        _tc_body,
        in_specs=[
            pl.BlockSpec(memory_space=pltpu.SMEM),
            pl.BlockSpec((_WPAD, _AB), lambda: (0, 0)),
            pl.BlockSpec((_AB, 1), lambda: (0, 0)),
        ],
        out_specs=[
            pl.BlockSpec((_WPAD, 16), lambda: (0, 0)),
            pl.BlockSpec((_WPAD, 16), lambda: (0, 0)),
        ],
        out_shape=[
            jax.ShapeDtypeStruct((_WPAD, 16), jnp.float32),
            jax.ShapeDtypeStruct((_WPAD, 16), jnp.float32),
        ],
    )(b, wf, W)
    out = _sc_body(tau, lp2b, db)
    return out.transpose(0, 2, 1)
